# baseline (speedup 1.0000x reference)
import jax
import jax.numpy as jnp
from jax import lax
from jax.experimental import pallas as pl
from jax.experimental.pallas import tpu as pltpu, tpu_sc as plsc

NC = 2
NS = 16
L = 16
NW = NC * NS
B = 16384
D = 32
BPW = B // NW
RPP = 256
PASSES = BPW // RPP
GPP = RPP // L
NSEM = 2

_MESH = plsc.VectorSubcoreMesh(core_axis_name="c", subcore_axis_name="s")


def _movies_gather_body(movies_hbm, memb_hbm, rows_hbm, midx_v, mrows_v, sem):
    wid = lax.axis_index("s") * NC + lax.axis_index("c")
    base = wid * BPW
    pltpu.sync_copy(movies_hbm.at[pl.ds(base, BPW)], midx_v)
    pltpu.async_copy(memb_hbm.at[midx_v], mrows_v, sem).wait()
    pltpu.sync_copy(mrows_v, rows_hbm.at[pl.ds(base, BPW), :])


def _dot_body(users_hbm, uemb_hbm, mrows_hbm, out_hbm,
              uidx_v, urows_v, mrows_v, out_v, *sems):
    wid = lax.axis_index("s") * NC + lax.axis_index("c")
    base = wid * BPW

    pltpu.sync_copy(users_hbm.at[pl.ds(base, BPW)], uidx_v)
    lane = lax.iota(jnp.int32, L)

    def one_pass(p, carry):
        pb = p * RPP

        def issue(g, c):
            uv = uidx_v[pl.ds(pb + g * L, L)]
            for j in range(L):
                i = uv[j]
                pltpu.make_async_copy(
                    uemb_hbm.at[i], urows_v.at[g * L + j],
                    sems[j % NSEM]).start()
            return c

        lax.fori_loop(0, GPP, issue, 0)
        pltpu.sync_copy(mrows_hbm.at[pl.ds(base + pb, RPP), :], mrows_v)
        per_sem_rows = RPP // NSEM
        for q in range(NSEM):
            pltpu.make_async_copy(
                uemb_hbm.at[pl.ds(0, per_sem_rows), :],
                urows_v.at[pl.ds(0, per_sem_rows), :], sems[q]).wait()

        def group(g, c):
            rows = g * L + lane
            acc = jnp.zeros((L,), jnp.float32)
            for d in range(D):
                col = jnp.full((L,), d, jnp.int32)
                u = plsc.load_gather(urows_v, [rows, col])
                m = plsc.load_gather(mrows_v, [rows, col])
                acc = acc + u * m
            out_v[pl.ds(pb + g * L, L)] = 1.0 + 9.0 / (1.0 + jnp.exp(-acc))
            return c

        lax.fori_loop(0, GPP, group, 0)
        return carry

    lax.fori_loop(0, PASSES, one_pass, 0)
    pltpu.sync_copy(out_v, out_hbm.at[pl.ds(base, BPW)])


@jax.jit
def kernel(users, movies, users_emb, movies_emb):
    users = users.astype(jnp.int32)
    movies = movies.astype(jnp.int32)

    gather_movies = pl.kernel(
        _movies_gather_body,
        out_type=jax.ShapeDtypeStruct((B, D), jnp.float32),
        mesh=_MESH,
        compiler_params=pltpu.CompilerParams(
            needs_layout_passes=False, use_tc_tiling_on_sc=False),
        scratch_types=[
            pltpu.VMEM((BPW,), jnp.int32),
            pltpu.VMEM((BPW, D), jnp.float32),
            pltpu.SemaphoreType.DMA,
        ],
    )
    mrows = gather_movies(movies, movies_emb)

    dot = pl.kernel(
        _dot_body,
        out_type=jax.ShapeDtypeStruct((B,), jnp.float32),
        mesh=_MESH,
        compiler_params=pltpu.CompilerParams(
            needs_layout_passes=False, use_tc_tiling_on_sc=True),
        scratch_types=[
            pltpu.VMEM((BPW,), jnp.int32),
            pltpu.VMEM((RPP, D), jnp.float32),
            pltpu.VMEM((RPP, D), jnp.float32),
            pltpu.VMEM((BPW,), jnp.float32),
        ] + [pltpu.SemaphoreType.DMA] * NSEM,
    )
    return dot(users, users_emb, mrows)


# speedup vs baseline: 1.0853x; 1.0853x over previous
import jax
import jax.numpy as jnp
from jax import lax
from jax.experimental import pallas as pl
from jax.experimental.pallas import tpu as pltpu, tpu_sc as plsc

NC = 2
NS = 16
L = 16
NW = NC * NS
B = 16384
D = 32
BPW = B // NW
RPP = 128
PASSES = BPW // RPP
GPP = RPP // L
NSEM = 2


def _sc_body(users_hbm, movies_hbm, uemb_hbm, memb_hbm, out_hbm,
             uidx_v, midx_v, urows0_v, mrows0_v, urows1_v, mrows1_v,
             out_v, *sems):
    wid = lax.axis_index("s") * NC + lax.axis_index("c")
    base = wid * BPW

    pltpu.sync_copy(users_hbm.at[pl.ds(base, BPW)], uidx_v)
    pltpu.sync_copy(movies_hbm.at[pl.ds(base, BPW)], midx_v)
    lane = lax.iota(jnp.int32, L)

    ubufs = [urows0_v, urows1_v]
    mbufs = [mrows0_v, mrows1_v]

    def issue(p, parity):
        pb = p * RPP
        urows_v, mrows_v = ubufs[parity], mbufs[parity]
        psems = sems[parity * NSEM:(parity + 1) * NSEM]

        def body(g, c):
            uv = uidx_v[pl.ds(pb + g * L, L)]
            mv = midx_v[pl.ds(pb + g * L, L)]
            for j in range(L):
                i = uv[j]
                pltpu.make_async_copy(
                    uemb_hbm.at[i], urows_v.at[g * L + j],
                    psems[j % NSEM]).start()
                k = mv[j]
                pltpu.make_async_copy(
                    memb_hbm.at[k], mrows_v.at[g * L + j],
                    psems[j % NSEM]).start()
            return c

        lax.fori_loop(0, GPP, body, 0)

    def drain_and_compute(p, parity):
        pb = p * RPP
        urows_v, mrows_v = ubufs[parity], mbufs[parity]
        psems = sems[parity * NSEM:(parity + 1) * NSEM]
        per_sem_rows = 2 * RPP // NSEM
        for q in range(NSEM):
            pltpu.make_async_copy(
                uemb_hbm.at[pl.ds(0, per_sem_rows), :],
                urows_v.at[pl.ds(0, per_sem_rows), :], psems[q]).wait()

        def group(g, c):
            rows = g * L + lane
            acc = jnp.zeros((L,), jnp.float32)
            for d in range(D):
                col = jnp.full((L,), d, jnp.int32)
                u = plsc.load_gather(urows_v, [rows, col])
                m = plsc.load_gather(mrows_v, [rows, col])
                acc = acc + u * m
            out_v[pl.ds(pb + g * L, L)] = 1.0 + 9.0 / (1.0 + jnp.exp(-acc))
            return c

        lax.fori_loop(0, GPP, group, 0)

    issue(0, 0)
    for p in range(1, PASSES):
        issue(p, p % 2)
        drain_and_compute(p - 1, (p - 1) % 2)
    drain_and_compute(PASSES - 1, (PASSES - 1) % 2)

    pltpu.sync_copy(out_v, out_hbm.at[pl.ds(base, BPW)])


@jax.jit
def kernel(users, movies, users_emb, movies_emb):
    users = users.astype(jnp.int32)
    movies = movies.astype(jnp.int32)
    f = pl.kernel(
        _sc_body,
        out_type=jax.ShapeDtypeStruct((B,), jnp.float32),
        mesh=plsc.VectorSubcoreMesh(core_axis_name="c", subcore_axis_name="s"),
        compiler_params=pltpu.CompilerParams(
            needs_layout_passes=False, use_tc_tiling_on_sc=True),
        scratch_types=[
            pltpu.VMEM((BPW,), jnp.int32),
            pltpu.VMEM((BPW,), jnp.int32),
            pltpu.VMEM((RPP, D), jnp.float32),
            pltpu.VMEM((RPP, D), jnp.float32),
            pltpu.VMEM((RPP, D), jnp.float32),
            pltpu.VMEM((RPP, D), jnp.float32),
            pltpu.VMEM((BPW,), jnp.float32),
        ] + [pltpu.SemaphoreType.DMA] * (2 * NSEM),
    )
    return f(users, movies, users_emb, movies_emb)
